# baseline (speedup 1.0000x reference)
import functools

import jax
import jax.numpy as jnp
from jax import lax
from jax.experimental import pallas as pl
from jax.experimental.pallas import tpu as pltpu
from jax.experimental.pallas import tpu_sc as plsc

HIDDEN = 4096
NUM_EXPERTS = 64
TOP_K = 8
TOKENS = 16384
BLOCK = 1024

SC_ROWS = 6144
SC_WORKERS = 32
SC_CHUNK = 8
_SC_PER_W = SC_ROWS // SC_WORKERS


def _make_sc_stream():
    mesh = plsc.VectorSubcoreMesh(core_axis_name="c", subcore_axis_name="s")

    @functools.partial(
        pl.kernel, mesh=mesh,
        out_type=jax.ShapeDtypeStruct((SC_WORKERS, 16), jnp.float32),
        scratch_types=[pltpu.VMEM((SC_CHUNK, HIDDEN), jnp.float32)],
    )
    def sc_stream(x_hbm, out_hbm, buf):
        wid = lax.axis_index("s") * 2 + lax.axis_index("c")
        base = wid * _SC_PER_W
        for it in range(_SC_PER_W // SC_CHUNK):
            pltpu.sync_copy(x_hbm.at[pl.ds(base + it * SC_CHUNK, SC_CHUNK)],
                            buf)
        pltpu.sync_copy(buf.at[0, pl.ds(0, 16)], out_hbm.at[wid])

    return sc_stream


def _router_body(x_ref, wt_ref, scores_ref, w_ref, idx_ref):
    logits = jax.lax.dot_general(
        x_ref[...], wt_ref[...],
        dimension_numbers=(((1,), (0,)), ((), ())),
        preferred_element_type=jnp.float32,
        precision=jax.lax.Precision.DEFAULT,
    )
    lt = logits.T
    m = jnp.max(lt, axis=0, keepdims=True)
    e = jnp.exp(lt - m)
    s = jnp.sum(e, axis=0, keepdims=True)
    scores_t = e / s
    scores_ref[...] = scores_t.T

    iota = jax.lax.broadcasted_iota(jnp.int32, scores_t.shape, 0)
    cur = scores_t
    vals = []
    idxs = []
    for _ in range(TOP_K):
        mx = jnp.max(cur, axis=0, keepdims=True)
        amx = jnp.min(jnp.where(cur == mx, iota, NUM_EXPERTS),
                      axis=0, keepdims=True)
        vals.append(mx)
        idxs.append(amx)
        cur = jnp.where(iota == amx, -1.0, cur)
    v = jnp.concatenate(vals, axis=0)
    ii = jnp.concatenate(idxs, axis=0)
    norm = jnp.sum(v, axis=0, keepdims=True)
    w_ref[...] = (v / norm).T
    idx_ref[...] = ii.T


def kernel(x, W):
    wt = W.T
    grid = (TOKENS // BLOCK,)
    scores, weights, top_experts = pl.pallas_call(
        _router_body,
        grid=grid,
        in_specs=[
            pl.BlockSpec((BLOCK, HIDDEN), lambda i: (i, 0)),
            pl.BlockSpec((HIDDEN, NUM_EXPERTS), lambda i: (0, 0)),
        ],
        out_specs=[
            pl.BlockSpec((BLOCK, NUM_EXPERTS), lambda i: (i, 0)),
            pl.BlockSpec((BLOCK, TOP_K), lambda i: (i, 0)),
            pl.BlockSpec((BLOCK, TOP_K), lambda i: (i, 0)),
        ],
        out_shape=[
            jax.ShapeDtypeStruct((TOKENS, NUM_EXPERTS), jnp.float32),
            jax.ShapeDtypeStruct((TOKENS, TOP_K), jnp.float32),
            jax.ShapeDtypeStruct((TOKENS, TOP_K), jnp.int32),
        ],
        compiler_params=pltpu.CompilerParams(
            dimension_semantics=("arbitrary",),
        ),
    )(x, wt)
    sc_out = _make_sc_stream()(x)
    big = jnp.max(jnp.abs(sc_out)) + jnp.float32(1e30)
    weights = jnp.minimum(weights, big)
    return (scores, weights, top_experts)


# speedup vs baseline: 1.4809x; 1.4809x over previous
import jax
import jax.numpy as jnp
from jax.experimental import pallas as pl
from jax.experimental.pallas import tpu as pltpu

HIDDEN = 4096
NUM_EXPERTS = 64
TOP_K = 8
TOKENS = 16384
BLOCK = 1024


def _router_body(x_ref, wt_ref, scores_ref, w_ref, idx_ref):
    logits = jax.lax.dot_general(
        x_ref[...], wt_ref[...],
        dimension_numbers=(((1,), (0,)), ((), ())),
        preferred_element_type=jnp.float32,
        precision=jax.lax.Precision.DEFAULT,
    )
    lt = logits.T
    m = jnp.max(lt, axis=0, keepdims=True)
    e = jnp.exp(lt - m)
    s = jnp.sum(e, axis=0, keepdims=True)
    scores_t = e / s
    scores_ref[...] = scores_t.T

    iota = jax.lax.broadcasted_iota(jnp.int32, scores_t.shape, 0)
    cur = scores_t
    vals = []
    idxs = []
    for _ in range(TOP_K):
        mx = jnp.max(cur, axis=0, keepdims=True)
        amx = jnp.min(jnp.where(cur == mx, iota, NUM_EXPERTS),
                      axis=0, keepdims=True)
        vals.append(mx)
        idxs.append(amx)
        cur = jnp.where(iota == amx, -1.0, cur)
    v = jnp.concatenate(vals, axis=0)
    ii = jnp.concatenate(idxs, axis=0)
    norm = jnp.sum(v, axis=0, keepdims=True)
    w_ref[...] = (v / norm).T
    idx_ref[...] = ii.T


def kernel(x, W):
    wt = W.T
    grid = (TOKENS // BLOCK,)
    scores, weights, top_experts = pl.pallas_call(
        _router_body,
        grid=grid,
        in_specs=[
            pl.BlockSpec((BLOCK, HIDDEN), lambda i: (i, 0)),
            pl.BlockSpec((HIDDEN, NUM_EXPERTS), lambda i: (0, 0)),
        ],
        out_specs=[
            pl.BlockSpec((BLOCK, NUM_EXPERTS), lambda i: (i, 0)),
            pl.BlockSpec((BLOCK, TOP_K), lambda i: (i, 0)),
            pl.BlockSpec((BLOCK, TOP_K), lambda i: (i, 0)),
        ],
        out_shape=[
            jax.ShapeDtypeStruct((TOKENS, NUM_EXPERTS), jnp.float32),
            jax.ShapeDtypeStruct((TOKENS, TOP_K), jnp.float32),
            jax.ShapeDtypeStruct((TOKENS, TOP_K), jnp.int32),
        ],
        compiler_params=pltpu.CompilerParams(
            dimension_semantics=("arbitrary",),
        ),
    )(x, wt)
    return (scores, weights, top_experts)
